# Initial kernel scaffold; baseline (speedup 1.0000x reference)
import jax
import jax.numpy as jnp
from jax.experimental import pallas as pl


def kernel(features, motion, w1a, b1a, g1a, be1a, w2a, b2a, g2a, be2a, w1b, b1b, g1b, be1b, w2b, b2b, g2b, be2b, delta):
    raise NotImplementedError("write your pallas kernel here")


# speedup vs baseline: 10.1539x; 10.1539x over previous
import functools

import jax
import jax.numpy as jnp
from jax import lax
from jax.experimental import pallas as pl
from jax.experimental.pallas import tpu as pltpu
from jax.experimental.pallas import tpu_sc as plsc

_B, _C, _N, _K = 8, 128, 2000, 9
_RT = 400
_NT = _N // _RT
_F32 = jnp.float32

def _topk_body(xt_ref, x_ref, o_ref, *, tap_off):
    b = pl.program_id(0)
    xt = xt_ref[0]
    x = x_ref[0]
    d = 2.0 * jnp.dot(xt, x, preferred_element_type=_F32)
    d = d - jnp.sum(x * x, axis=0, keepdims=True)
    d = d - jnp.sum(xt * xt, axis=1, keepdims=True)
    lane = lax.broadcasted_iota(jnp.int32, d.shape, 1)
    lane_k = lax.broadcasted_iota(jnp.int32, (_RT, _K), 1)
    out = jnp.zeros((_RT, _K), jnp.int32)
    for kk in range(_K):
        m = jnp.max(d, axis=1, keepdims=True)
        cand = jnp.where(d >= m, lane, _N + 7)
        am = jnp.min(cand, axis=1, keepdims=True)
        d = jnp.where(lane == am, -jnp.inf, d)
        out = jnp.where(lane_k == kk, am, out)
    tap = lane_k % 3 + tap_off
    o_ref[0] = out * 6 + (b * _N * 6 + tap)


def _topk_call(xt, x, cp, tap_off):
    return pl.pallas_call(
        functools.partial(_topk_body, tap_off=tap_off),
        grid=(_B, _NT),
        in_specs=[
            pl.BlockSpec((1, _RT, cp), lambda b, i: (b, i, 0)),
            pl.BlockSpec((1, cp, _N), lambda b, i: (b, 0, 0)),
        ],
        out_specs=pl.BlockSpec((1, _RT, _K), lambda b, i: (b, i, 0)),
        out_shape=jax.ShapeDtypeStruct((_B, _N, _K), jnp.int32),
    )(xt, x)


def _tables_body(xt_ref, w_ref, tbl_ref, y1_ref):
    res = jnp.dot(xt_ref[0], w_ref[...], preferred_element_type=_F32)
    tbl_ref[0] = res[:, :768]
    y1_ref[0] = res[:, 768:]


def _tables_call(xt, wc):
    return pl.pallas_call(
        _tables_body,
        grid=(_B, _NT),
        in_specs=[
            pl.BlockSpec((1, _RT, _C), lambda b, i: (b, i, 0)),
            pl.BlockSpec((_C, 1024), lambda b, i: (0, 0)),
        ],
        out_specs=[
            pl.BlockSpec((1, _RT, 768), lambda b, i: (b, i, 0)),
            pl.BlockSpec((1, _RT, 256), lambda b, i: (b, i, 0)),
        ],
        out_shape=[
            jax.ShapeDtypeStruct((_B, _N, 768), _F32),
            jax.ShapeDtypeStruct((_B, _N, 256), _F32),
        ],
    )(xt, wc)


_GTOT = 2 * _B * _N * _K
_NW = 32
_GW = _GTOT // _NW
_GCH = 120
_GNC = _GW // _GCH


def _gather_body(tbl_hbm, idx_hbm, out_hbm, idx_v, rows_v, sem):
    c = lax.axis_index("c")
    s = lax.axis_index("s")
    wid = s * 2 + c
    pltpu.sync_copy(idx_hbm.at[wid], idx_v)

    def chunk(j, carry):
        pltpu.async_copy(tbl_hbm.at[idx_v.at[j]], rows_v, sem).wait()
        pltpu.sync_copy(rows_v, out_hbm.at[pl.ds(wid * _GW + j * _GCH, _GCH)])
        return carry

    lax.fori_loop(0, _GNC, chunk, 0)


_gather_rows = pl.kernel(
    _gather_body,
    out_type=jax.ShapeDtypeStruct((_GTOT, _C), _F32),
    mesh=plsc.VectorSubcoreMesh(core_axis_name="c", subcore_axis_name="s"),
    scratch_types=[
        pltpu.VMEM((_GNC, _GCH), jnp.int32),
        pltpu.VMEM((_GCH, _C), _F32),
        pltpu.SemaphoreType.DMA,
    ],
)


def _h_body(y1_ref, g_ref, b1_ref, h_ref, st_ref):
    b = pl.program_id(1)
    i = pl.program_id(2)
    base = y1_ref[0] + b1_ref[0]
    g = g_ref[0, 0]
    s1 = jnp.zeros((1, _C), _F32)
    s2 = jnp.zeros((1, _C), _F32)
    for j in range(3):
        hj = base - (g[:, 3 * j] + g[:, 3 * j + 1] + g[:, 3 * j + 2])
        h_ref[0, 0, :, j] = hj
        s1 = s1 + jnp.sum(hj, axis=0, keepdims=True)
        s2 = s2 + jnp.sum(hj * hj, axis=0, keepdims=True)
    acc = jnp.concatenate([s1, s2], axis=0)

    @pl.when(jnp.logical_and(b == 0, i == 0))
    def _():
        st_ref[0] = acc

    @pl.when(jnp.logical_or(b > 0, i > 0))
    def _():
        st_ref[0] = st_ref[0] + acc


def _h_call(y1, g, b1s):
    return pl.pallas_call(
        _h_body,
        grid=(2, _B, _NT),
        in_specs=[
            pl.BlockSpec((1, _RT, _C), lambda r, b, i: (b, i, r)),
            pl.BlockSpec((1, 1, _RT, _K, _C), lambda r, b, i: (r, b, i, 0, 0)),
            pl.BlockSpec((1, 1, _C), lambda r, b, i: (r, 0, 0)),
        ],
        out_specs=[
            pl.BlockSpec((1, 1, _RT, 3, _C), lambda r, b, i: (r, b, i, 0, 0)),
            pl.BlockSpec((1, 2, _C), lambda r, b, i: (r, 0, 0)),
        ],
        out_shape=[
            jax.ShapeDtypeStruct((2, _B, _N, 3, _C), _F32),
            jax.ShapeDtypeStruct((2, 2, _C), _F32),
        ],
    )(y1, g, b1s)


def _conv2_body(h_ref, sc_ref, sh_ref, w_ref, b2_ref, h2_ref, st_ref):
    b = pl.program_id(1)
    i = pl.program_id(2)
    h = h_ref[0, 0]
    scale = sc_ref[0]
    shift = sh_ref[0]
    tot = jnp.zeros((_RT, _C), _F32)
    for j in range(3):
        r = jnp.maximum(h[:, j] * scale + shift, 0.0)
        tot = tot + jnp.dot(r, w_ref[0, j], preferred_element_type=_F32)
    h2 = tot + b2_ref[0]
    h2_ref[0, 0] = h2
    s1 = jnp.sum(h2, axis=0, keepdims=True)
    s2 = jnp.sum(h2 * h2, axis=0, keepdims=True)
    acc = jnp.concatenate([s1, s2], axis=0)

    @pl.when(jnp.logical_and(b == 0, i == 0))
    def _():
        st_ref[0] = acc

    @pl.when(jnp.logical_or(b > 0, i > 0))
    def _():
        st_ref[0] = st_ref[0] + acc


def _conv2_call(h, sc1, sh1, w2c, b2s):
    return pl.pallas_call(
        _conv2_body,
        grid=(2, _B, _NT),
        in_specs=[
            pl.BlockSpec((1, 1, _RT, 3, _C), lambda r, b, i: (r, b, i, 0, 0)),
            pl.BlockSpec((1, 1, _C), lambda r, b, i: (r, 0, 0)),
            pl.BlockSpec((1, 1, _C), lambda r, b, i: (r, 0, 0)),
            pl.BlockSpec((1, 3, _C, _C), lambda r, b, i: (r, 0, 0, 0)),
            pl.BlockSpec((1, 1, _C), lambda r, b, i: (r, 0, 0)),
        ],
        out_specs=[
            pl.BlockSpec((1, 1, _RT, _C), lambda r, b, i: (r, b, i, 0)),
            pl.BlockSpec((1, 2, _C), lambda r, b, i: (r, 0, 0)),
        ],
        out_shape=[
            jax.ShapeDtypeStruct((2, _B, _N, _C), _F32),
            jax.ShapeDtypeStruct((2, 2, _C), _F32),
        ],
    )(h, sc1, sh1, w2c, b2s)


def _final_body(h2_ref, p_ref, d_ref, o_ref):
    r1 = jnp.maximum(h2_ref[0, 0] * p_ref[0:1] + p_ref[1:2], 0.0)
    r2 = jnp.maximum(h2_ref[1, 0] * p_ref[2:3] + p_ref[3:4], 0.0)
    res = r1 + d_ref[0, 0] * r2
    o_ref[0] = res.T


def _final_call(h2, params, delta):
    return pl.pallas_call(
        _final_body,
        grid=(_B,),
        in_specs=[
            pl.BlockSpec((2, 1, _N, _C), lambda b: (0, b, 0, 0)),
            pl.BlockSpec((4, _C), lambda b: (0, 0)),
            pl.BlockSpec((1, 1), lambda b: (0, 0)),
        ],
        out_specs=pl.BlockSpec((1, _C, _N), lambda b: (b, 0, 0)),
        out_shape=jax.ShapeDtypeStruct((_B, _C, _N), _F32),
    )(h2, params, delta)


def kernel(features, motion, w1a, b1a, g1a, be1a, w2a, b2a, g2a, be2a,
           w1b, b1b, g1b, be1b, w2b, b2b, g2b, be2b, delta):
    xf = features.reshape(_B, _C, _N)
    xm = motion.reshape(_B, -1, _N)
    cm = xm.shape[1]
    xft = jnp.swapaxes(xf, 1, 2)
    xm8 = jnp.concatenate(
        [xm, jnp.zeros((_B, 8 - cm, _N), _F32)], axis=1)
    xmt = jnp.swapaxes(xm8, 1, 2)

    idx_a = _topk_call(xft, xf, _C, 0)
    idx_b = _topk_call(xmt, xm8, 8, 3)

    w1a_, w1b_ = w1a[:, :, 0, :], w1b[:, :, 0, :]
    taps = [w1a_[:, _C:, t].T for t in range(3)]
    taps += [w1b_[:, _C:, t].T for t in range(3)]
    a_a = jnp.sum(w1a_[:, :_C, :] + w1a_[:, _C:, :], axis=2).T
    a_b = jnp.sum(w1b_[:, :_C, :] + w1b_[:, _C:, :], axis=2).T
    wc = jnp.concatenate(taps + [a_a, a_b], axis=1)

    tbl, y1 = _tables_call(xft, wc)
    tblf = tbl.reshape(_B * _N * 6, _C)
    idx2 = jnp.concatenate(
        [idx_a.reshape(-1), idx_b.reshape(-1)]).reshape(_NW, _GNC, _GCH)

    g = _gather_rows(tblf, idx2).reshape(2, _B, _N, _K, _C)

    b1s = jnp.stack([b1a, b1b]).reshape(2, 1, _C)
    h, st1 = _h_call(y1, g, b1s)

    m1 = float(_B * _N * 3)
    mu1 = st1[:, 0] / m1
    var1 = st1[:, 1] / m1 - mu1 * mu1
    g1s = jnp.stack([g1a, g1b])
    be1s = jnp.stack([be1a, be1b])
    sc1 = g1s * lax.rsqrt(var1 + 1e-5)
    sh1 = be1s - mu1 * sc1

    w2c = jnp.stack([jnp.transpose(w2a[:, :, 0, :], (2, 1, 0)),
                     jnp.transpose(w2b[:, :, 0, :], (2, 1, 0))])
    b2s = jnp.stack([b2a, b2b]).reshape(2, 1, _C)
    h2, st2 = _conv2_call(h, sc1.reshape(2, 1, _C), sh1.reshape(2, 1, _C),
                          w2c, b2s)

    m2 = float(_B * _N)
    mu2 = st2[:, 0] / m2
    var2 = st2[:, 1] / m2 - mu2 * mu2
    g2s = jnp.stack([g2a, g2b])
    be2s = jnp.stack([be2a, be2b])
    sc2 = g2s * lax.rsqrt(var2 + 1e-5)
    sh2 = be2s - mu2 * sc2
    params = jnp.stack([sc2[0], sh2[0], sc2[1], sh2[1]])

    out = _final_call(h2, params, delta.reshape(1, 1))
    return out.reshape(_B, _C, _N, 1)


# speedup vs baseline: 12.4873x; 1.2298x over previous
import functools

import jax
import jax.numpy as jnp
from jax import lax
from jax.experimental import pallas as pl
from jax.experimental.pallas import tpu as pltpu
from jax.experimental.pallas import tpu_sc as plsc

_B, _C, _N, _K = 8, 128, 2000, 9
_RT = 400
_NT = _N // _RT
_F32 = jnp.float32

def _topk_body(xt_ref, x_ref, o_ref, *, tap_off):
    b = pl.program_id(0)
    xt = xt_ref[0]
    x = x_ref[0]
    d = 2.0 * jnp.dot(xt, x, preferred_element_type=_F32)
    d = d - jnp.sum(x * x, axis=0, keepdims=True)
    d = d - jnp.sum(xt * xt, axis=1, keepdims=True)
    lane = lax.broadcasted_iota(jnp.int32, d.shape, 1)
    lane_k = lax.broadcasted_iota(jnp.int32, (_RT, _K), 1)
    out = jnp.zeros((_RT, _K), jnp.int32)
    for kk in range(_K):
        m = jnp.max(d, axis=1, keepdims=True)
        cand = jnp.where(d >= m, lane, _N + 7)
        am = jnp.min(cand, axis=1, keepdims=True)
        d = jnp.where(lane == am, -jnp.inf, d)
        out = jnp.where(lane_k == kk, am, out)
    tap = (lane_k % 3 + tap_off) * (_B * _N)
    o_ref[0] = out + (tap + b * _N)


def _topk_call(xt, x, cp, tap_off):
    return pl.pallas_call(
        functools.partial(_topk_body, tap_off=tap_off),
        grid=(_B, _NT),
        in_specs=[
            pl.BlockSpec((1, _RT, cp), lambda b, i: (b, i, 0)),
            pl.BlockSpec((1, cp, _N), lambda b, i: (b, 0, 0)),
        ],
        out_specs=pl.BlockSpec((1, _RT, _K), lambda b, i: (b, i, 0)),
        out_shape=jax.ShapeDtypeStruct((_B, _N, _K), jnp.int32),
    )(xt, x)


def _tables_body(xt_ref, w_ref, tbl_ref, y1_ref):
    res = jnp.dot(xt_ref[0], w_ref[...], preferred_element_type=_F32)
    for t in range(6):
        tbl_ref[t, 0] = res[:, t * _C:(t + 1) * _C]
    y1_ref[0, 0] = res[:, 768:896]
    y1_ref[1, 0] = res[:, 896:]


def _tables_call(xt, wc):
    return pl.pallas_call(
        _tables_body,
        grid=(_B, _NT),
        in_specs=[
            pl.BlockSpec((1, _RT, _C), lambda b, i: (b, i, 0)),
            pl.BlockSpec((_C, 1024), lambda b, i: (0, 0)),
        ],
        out_specs=[
            pl.BlockSpec((6, 1, _RT, _C), lambda b, i: (0, b, i, 0)),
            pl.BlockSpec((2, 1, _RT, _C), lambda b, i: (0, b, i, 0)),
        ],
        out_shape=[
            jax.ShapeDtypeStruct((6, _B, _N, _C), _F32),
            jax.ShapeDtypeStruct((2, _B, _N, _C), _F32),
        ],
    )(xt, wc)


_NPOS = 2 * _B * _N
_NW = 32
_PW = _NPOS // _NW
_PCH = 40
_CNC = _PW // _PCH
_GROWS = _PCH * _K
_IROW = 120


def _gather_body(tbl_hbm, idx_hbm, out_hbm, idx_v, rows_v, out_v, sem):
    c = lax.axis_index("c")
    s = lax.axis_index("s")
    wid = s * 2 + c
    pltpu.sync_copy(idx_hbm.at[wid], idx_v)

    def chunk(ci, carry):
        cps = []
        for q in range(3):
            cps.append(pltpu.async_copy(
                tbl_hbm.at[idx_v.at[ci * 3 + q]],
                rows_v.at[pl.ds(q * _IROW, _IROW)], sem))
        for cp in cps:
            cp.wait()

        def pos(nl, acc):
            base = nl * _K
            for j in range(3):
                r0 = base + 3 * j
                for cc in range(8):
                    sl = pl.ds(cc * 16, 16)
                    out_v[j, nl, sl] = (rows_v[r0, sl] + rows_v[r0 + 1, sl]
                                        + rows_v[r0 + 2, sl])
            return acc

        lax.fori_loop(0, _PCH, pos, 0)
        p0 = wid * _PW + ci * _PCH
        obase = (p0 // _N) * (3 * _N) + (p0 % _N)
        for j in range(3):
            pltpu.sync_copy(out_v.at[j],
                            out_hbm.at[pl.ds(obase + j * _N, _PCH)])
        return carry

    lax.fori_loop(0, _CNC, chunk, 0)


_gather_rows = pl.kernel(
    _gather_body,
    out_type=jax.ShapeDtypeStruct((2 * _B * 3 * _N, _C), _F32),
    mesh=plsc.VectorSubcoreMesh(core_axis_name="c", subcore_axis_name="s"),
    scratch_types=[
        pltpu.VMEM((_NPOS * _K // _NW // _IROW, _IROW), jnp.int32),
        pltpu.VMEM((_GROWS, _C), _F32),
        pltpu.VMEM((3, _PCH, _C), _F32),
        pltpu.SemaphoreType.DMA,
    ],
)


def _h_body(y1_ref, g_ref, b1_ref, h_ref, st_ref):
    b = pl.program_id(1)
    i = pl.program_id(2)
    base = y1_ref[0, 0] + b1_ref[0]
    g = g_ref[0, 0]
    s1 = jnp.zeros((1, _C), _F32)
    s2 = jnp.zeros((1, _C), _F32)
    for j in range(3):
        hj = base - g[j]
        h_ref[0, 0, j] = hj
        s1 = s1 + jnp.sum(hj, axis=0, keepdims=True)
        s2 = s2 + jnp.sum(hj * hj, axis=0, keepdims=True)
    acc = jnp.concatenate([s1, s2], axis=0)

    @pl.when(jnp.logical_and(b == 0, i == 0))
    def _():
        st_ref[0] = acc

    @pl.when(jnp.logical_or(b > 0, i > 0))
    def _():
        st_ref[0] = st_ref[0] + acc


def _h_call(y1, g, b1s):
    return pl.pallas_call(
        _h_body,
        grid=(2, _B, _NT),
        in_specs=[
            pl.BlockSpec((1, 1, _RT, _C), lambda r, b, i: (r, b, i, 0)),
            pl.BlockSpec((1, 1, 3, _RT, _C), lambda r, b, i: (r, b, 0, i, 0)),
            pl.BlockSpec((1, 1, _C), lambda r, b, i: (r, 0, 0)),
        ],
        out_specs=[
            pl.BlockSpec((1, 1, 3, _RT, _C), lambda r, b, i: (r, b, 0, i, 0)),
            pl.BlockSpec((1, 2, _C), lambda r, b, i: (r, 0, 0)),
        ],
        out_shape=[
            jax.ShapeDtypeStruct((2, _B, 3, _N, _C), _F32),
            jax.ShapeDtypeStruct((2, 2, _C), _F32),
        ],
    )(y1, g, b1s)


def _conv2_body(h_ref, sc_ref, sh_ref, w_ref, b2_ref, h2_ref, st_ref):
    b = pl.program_id(1)
    i = pl.program_id(2)
    scale = sc_ref[0]
    shift = sh_ref[0]
    tot = b2_ref[0] + jnp.zeros((_RT, _C), _F32)
    for j in range(3):
        r = jnp.maximum(h_ref[0, 0, j] * scale + shift, 0.0)
        tot = tot + jnp.dot(r, w_ref[0, j], preferred_element_type=_F32)
    h2_ref[0, 0] = tot
    s1 = jnp.sum(tot, axis=0, keepdims=True)
    s2 = jnp.sum(tot * tot, axis=0, keepdims=True)
    acc = jnp.concatenate([s1, s2], axis=0)

    @pl.when(jnp.logical_and(b == 0, i == 0))
    def _():
        st_ref[0] = acc

    @pl.when(jnp.logical_or(b > 0, i > 0))
    def _():
        st_ref[0] = st_ref[0] + acc


def _conv2_call(h, sc1, sh1, w2c, b2s):
    return pl.pallas_call(
        _conv2_body,
        grid=(2, _B, _NT),
        in_specs=[
            pl.BlockSpec((1, 1, 3, _RT, _C), lambda r, b, i: (r, b, 0, i, 0)),
            pl.BlockSpec((1, 1, _C), lambda r, b, i: (r, 0, 0)),
            pl.BlockSpec((1, 1, _C), lambda r, b, i: (r, 0, 0)),
            pl.BlockSpec((1, 3, _C, _C), lambda r, b, i: (r, 0, 0, 0)),
            pl.BlockSpec((1, 1, _C), lambda r, b, i: (r, 0, 0)),
        ],
        out_specs=[
            pl.BlockSpec((1, 1, _RT, _C), lambda r, b, i: (r, b, i, 0)),
            pl.BlockSpec((1, 2, _C), lambda r, b, i: (r, 0, 0)),
        ],
        out_shape=[
            jax.ShapeDtypeStruct((2, _B, _N, _C), _F32),
            jax.ShapeDtypeStruct((2, 2, _C), _F32),
        ],
    )(h, sc1, sh1, w2c, b2s)


def _final_body(h2_ref, p_ref, d_ref, o_ref):
    r1 = jnp.maximum(h2_ref[0, 0] * p_ref[0:1] + p_ref[1:2], 0.0)
    r2 = jnp.maximum(h2_ref[1, 0] * p_ref[2:3] + p_ref[3:4], 0.0)
    res = r1 + d_ref[0, 0] * r2
    o_ref[0] = res.T


def _final_call(h2, params, delta):
    return pl.pallas_call(
        _final_body,
        grid=(_B,),
        in_specs=[
            pl.BlockSpec((2, 1, _N, _C), lambda b: (0, b, 0, 0)),
            pl.BlockSpec((4, _C), lambda b: (0, 0)),
            pl.BlockSpec((1, 1), lambda b: (0, 0)),
        ],
        out_specs=pl.BlockSpec((1, _C, _N), lambda b: (b, 0, 0)),
        out_shape=jax.ShapeDtypeStruct((_B, _C, _N), _F32),
    )(h2, params, delta)


def kernel(features, motion, w1a, b1a, g1a, be1a, w2a, b2a, g2a, be2a,
           w1b, b1b, g1b, be1b, w2b, b2b, g2b, be2b, delta):
    xf = features.reshape(_B, _C, _N)
    xm = motion.reshape(_B, -1, _N)
    cm = xm.shape[1]
    xft = jnp.swapaxes(xf, 1, 2)
    xm8 = jnp.concatenate(
        [xm, jnp.zeros((_B, 8 - cm, _N), _F32)], axis=1)
    xmt = jnp.swapaxes(xm8, 1, 2)

    idx_a = _topk_call(xft, xf, _C, 0)
    idx_b = _topk_call(xmt, xm8, 8, 3)

    w1a_, w1b_ = w1a[:, :, 0, :], w1b[:, :, 0, :]
    taps = [w1a_[:, _C:, t].T for t in range(3)]
    taps += [w1b_[:, _C:, t].T for t in range(3)]
    a_a = jnp.sum(w1a_[:, :_C, :] + w1a_[:, _C:, :], axis=2).T
    a_b = jnp.sum(w1b_[:, :_C, :] + w1b_[:, _C:, :], axis=2).T
    wc = jnp.concatenate(taps + [a_a, a_b], axis=1)

    tbl, y1 = _tables_call(xft, wc)
    tblf = tbl.reshape(6 * _B * _N, _C)
    idx2 = jnp.concatenate(
        [idx_a.reshape(-1), idx_b.reshape(-1)]).reshape(_NW, -1, _IROW)

    g = _gather_rows(tblf, idx2).reshape(2, _B, 3, _N, _C)

    b1s = jnp.stack([b1a, b1b]).reshape(2, 1, _C)
    h, st1 = _h_call(y1, g, b1s)

    m1 = float(_B * _N * 3)
    mu1 = st1[:, 0] / m1
    var1 = st1[:, 1] / m1 - mu1 * mu1
    g1s = jnp.stack([g1a, g1b])
    be1s = jnp.stack([be1a, be1b])
    sc1 = g1s * lax.rsqrt(var1 + 1e-5)
    sh1 = be1s - mu1 * sc1

    w2c = jnp.stack([jnp.transpose(w2a[:, :, 0, :], (2, 1, 0)),
                     jnp.transpose(w2b[:, :, 0, :], (2, 1, 0))])
    b2s = jnp.stack([b2a, b2b]).reshape(2, 1, _C)
    h2, st2 = _conv2_call(h, sc1.reshape(2, 1, _C), sh1.reshape(2, 1, _C),
                          w2c, b2s)

    m2 = float(_B * _N)
    mu2 = st2[:, 0] / m2
    var2 = st2[:, 1] / m2 - mu2 * mu2
    g2s = jnp.stack([g2a, g2b])
    be2s = jnp.stack([be2a, be2b])
    sc2 = g2s * lax.rsqrt(var2 + 1e-5)
    sh2 = be2s - mu2 * sc2
    params = jnp.stack([sc2[0], sh2[0], sc2[1], sh2[1]])

    out = _final_call(h2, params, delta.reshape(1, 1))
    return out.reshape(_B, _C, _N, 1)


# speedup vs baseline: 14.4826x; 1.1598x over previous
import functools

import jax
import jax.numpy as jnp
from jax import lax
from jax.experimental import pallas as pl
from jax.experimental.pallas import tpu as pltpu
from jax.experimental.pallas import tpu_sc as plsc

_B, _C, _N, _K = 8, 128, 2000, 9
_RT = 400
_NT = _N // _RT
_F32 = jnp.float32

def _topk_body(xt_ref, x_ref, o_ref, *, tap_off):
    b = pl.program_id(0)
    xt = xt_ref[0]
    x = x_ref[0]
    d = jnp.dot(xt + xt, x, preferred_element_type=_F32)
    d = d - jnp.sum(x * x, axis=0, keepdims=True)
    lane_f = lax.broadcasted_iota(jnp.int32, d.shape, 1).astype(_F32)
    lane_k = lax.broadcasted_iota(jnp.int32, (_RT, _K), 1)
    out = jnp.zeros((_RT, _K), jnp.int32)
    for kk in range(_K):
        m = jnp.max(d, axis=1, keepdims=True)
        ge = d >= m
        amf = jnp.min(jnp.where(ge, lane_f, float(_N + 7)), axis=1,
                      keepdims=True)
        d = jnp.where(ge, -jnp.inf, d)
        out = jnp.where(lane_k == kk, amf.astype(jnp.int32), out)
    tap = (lane_k % 3 + tap_off) * (_B * _N)
    o_ref[0] = out + (tap + b * _N)


def _topk_call(xt, x, cp, tap_off):
    return pl.pallas_call(
        functools.partial(_topk_body, tap_off=tap_off),
        grid=(_B, _NT),
        in_specs=[
            pl.BlockSpec((1, _RT, cp), lambda b, i: (b, i, 0)),
            pl.BlockSpec((1, cp, _N), lambda b, i: (b, 0, 0)),
        ],
        out_specs=pl.BlockSpec((1, _RT, _K), lambda b, i: (b, i, 0)),
        out_shape=jax.ShapeDtypeStruct((_B, _N, _K), jnp.int32),
    )(xt, x)


def _tables_body(xt_ref, w_ref, tbl_ref, y1_ref):
    res = jnp.dot(xt_ref[0], w_ref[...], preferred_element_type=_F32)
    for t in range(6):
        tbl_ref[t, 0] = res[:, t * _C:(t + 1) * _C]
    y1_ref[0, 0] = res[:, 768:896]
    y1_ref[1, 0] = res[:, 896:]


def _tables_call(xt, wc):
    return pl.pallas_call(
        _tables_body,
        grid=(_B, _NT),
        in_specs=[
            pl.BlockSpec((1, _RT, _C), lambda b, i: (b, i, 0)),
            pl.BlockSpec((_C, 1024), lambda b, i: (0, 0)),
        ],
        out_specs=[
            pl.BlockSpec((6, 1, _RT, _C), lambda b, i: (0, b, i, 0)),
            pl.BlockSpec((2, 1, _RT, _C), lambda b, i: (0, b, i, 0)),
        ],
        out_shape=[
            jax.ShapeDtypeStruct((6, _B, _N, _C), _F32),
            jax.ShapeDtypeStruct((2, _B, _N, _C), _F32),
        ],
    )(xt, wc)


_NPOS = 2 * _B * _N
_NW = 32
_PW = _NPOS // _NW
_PCH = 40
_CNC = _PW // _PCH
_GROWS = _PCH * _K
_IROW = 120


def _gather_body(tbl_hbm, idx_hbm, out_hbm, idx_v, rows_v, out_v, sem):
    c = lax.axis_index("c")
    s = lax.axis_index("s")
    wid = s * 2 + c
    pltpu.sync_copy(idx_hbm.at[wid], idx_v)

    def chunk(ci, carry):
        cps = []
        for q in range(3):
            cps.append(pltpu.async_copy(
                tbl_hbm.at[idx_v.at[ci * 3 + q]],
                rows_v.at[pl.ds(q * _IROW, _IROW)], sem))
        for cp in cps:
            cp.wait()

        def pos(nl, acc):
            base = nl * _K
            for j in range(3):
                r0 = base + 3 * j
                for cc in range(8):
                    sl = pl.ds(cc * 16, 16)
                    out_v[j, nl, sl] = (rows_v[r0, sl] + rows_v[r0 + 1, sl]
                                        + rows_v[r0 + 2, sl])
            return acc

        lax.fori_loop(0, _PCH, pos, 0)
        p0 = wid * _PW + ci * _PCH
        obase = (p0 // _N) * (3 * _N) + (p0 % _N)
        for j in range(3):
            pltpu.sync_copy(out_v.at[j],
                            out_hbm.at[pl.ds(obase + j * _N, _PCH)])
        return carry

    lax.fori_loop(0, _CNC, chunk, 0)


_gather_rows = pl.kernel(
    _gather_body,
    out_type=jax.ShapeDtypeStruct((2 * _B * 3 * _N, _C), _F32),
    mesh=plsc.VectorSubcoreMesh(core_axis_name="c", subcore_axis_name="s"),
    scratch_types=[
        pltpu.VMEM((_NPOS * _K // _NW // _IROW, _IROW), jnp.int32),
        pltpu.VMEM((_GROWS, _C), _F32),
        pltpu.VMEM((3, _PCH, _C), _F32),
        pltpu.SemaphoreType.DMA,
    ],
)


def _h_body(y1_ref, g_ref, b1_ref, h_ref, st_ref):
    b = pl.program_id(1)
    i = pl.program_id(2)
    base = y1_ref[0, 0] + b1_ref[0]
    g = g_ref[0, 0]
    s1 = jnp.zeros((1, _C), _F32)
    s2 = jnp.zeros((1, _C), _F32)
    for j in range(3):
        hj = base - g[j]
        h_ref[0, 0, j] = hj
        s1 = s1 + jnp.sum(hj, axis=0, keepdims=True)
        s2 = s2 + jnp.sum(hj * hj, axis=0, keepdims=True)
    acc = jnp.concatenate([s1, s2], axis=0)

    @pl.when(jnp.logical_and(b == 0, i == 0))
    def _():
        st_ref[0] = acc

    @pl.when(jnp.logical_or(b > 0, i > 0))
    def _():
        st_ref[0] = st_ref[0] + acc


def _h_call(y1, g, b1s):
    return pl.pallas_call(
        _h_body,
        grid=(2, _B, _NT),
        in_specs=[
            pl.BlockSpec((1, 1, _RT, _C), lambda r, b, i: (r, b, i, 0)),
            pl.BlockSpec((1, 1, 3, _RT, _C), lambda r, b, i: (r, b, 0, i, 0)),
            pl.BlockSpec((1, 1, _C), lambda r, b, i: (r, 0, 0)),
        ],
        out_specs=[
            pl.BlockSpec((1, 1, 3, _RT, _C), lambda r, b, i: (r, b, 0, i, 0)),
            pl.BlockSpec((1, 2, _C), lambda r, b, i: (r, 0, 0)),
        ],
        out_shape=[
            jax.ShapeDtypeStruct((2, _B, 3, _N, _C), _F32),
            jax.ShapeDtypeStruct((2, 2, _C), _F32),
        ],
    )(y1, g, b1s)


def _conv2_body(h_ref, sc_ref, sh_ref, w_ref, b2_ref, h2_ref, st_ref):
    b = pl.program_id(1)
    i = pl.program_id(2)
    scale = sc_ref[0]
    shift = sh_ref[0]
    tot = b2_ref[0] + jnp.zeros((_RT, _C), _F32)
    for j in range(3):
        r = jnp.maximum(h_ref[0, 0, j] * scale + shift, 0.0)
        tot = tot + jnp.dot(r, w_ref[0, j], preferred_element_type=_F32)
    h2_ref[0, 0] = tot
    s1 = jnp.sum(tot, axis=0, keepdims=True)
    s2 = jnp.sum(tot * tot, axis=0, keepdims=True)
    acc = jnp.concatenate([s1, s2], axis=0)

    @pl.when(jnp.logical_and(b == 0, i == 0))
    def _():
        st_ref[0] = acc

    @pl.when(jnp.logical_or(b > 0, i > 0))
    def _():
        st_ref[0] = st_ref[0] + acc


def _conv2_call(h, sc1, sh1, w2c, b2s):
    return pl.pallas_call(
        _conv2_body,
        grid=(2, _B, _NT),
        in_specs=[
            pl.BlockSpec((1, 1, 3, _RT, _C), lambda r, b, i: (r, b, 0, i, 0)),
            pl.BlockSpec((1, 1, _C), lambda r, b, i: (r, 0, 0)),
            pl.BlockSpec((1, 1, _C), lambda r, b, i: (r, 0, 0)),
            pl.BlockSpec((1, 3, _C, _C), lambda r, b, i: (r, 0, 0, 0)),
            pl.BlockSpec((1, 1, _C), lambda r, b, i: (r, 0, 0)),
        ],
        out_specs=[
            pl.BlockSpec((1, 1, _RT, _C), lambda r, b, i: (r, b, i, 0)),
            pl.BlockSpec((1, 2, _C), lambda r, b, i: (r, 0, 0)),
        ],
        out_shape=[
            jax.ShapeDtypeStruct((2, _B, _N, _C), _F32),
            jax.ShapeDtypeStruct((2, 2, _C), _F32),
        ],
    )(h, sc1, sh1, w2c, b2s)


def _final_body(h2_ref, p_ref, d_ref, o_ref):
    r1 = jnp.maximum(h2_ref[0, 0] * p_ref[0:1] + p_ref[1:2], 0.0)
    r2 = jnp.maximum(h2_ref[1, 0] * p_ref[2:3] + p_ref[3:4], 0.0)
    res = r1 + d_ref[0, 0] * r2
    o_ref[0] = res.T


def _final_call(h2, params, delta):
    return pl.pallas_call(
        _final_body,
        grid=(_B,),
        in_specs=[
            pl.BlockSpec((2, 1, _N, _C), lambda b: (0, b, 0, 0)),
            pl.BlockSpec((4, _C), lambda b: (0, 0)),
            pl.BlockSpec((1, 1), lambda b: (0, 0)),
        ],
        out_specs=pl.BlockSpec((1, _C, _N), lambda b: (b, 0, 0)),
        out_shape=jax.ShapeDtypeStruct((_B, _C, _N), _F32),
    )(h2, params, delta)


def kernel(features, motion, w1a, b1a, g1a, be1a, w2a, b2a, g2a, be2a,
           w1b, b1b, g1b, be1b, w2b, b2b, g2b, be2b, delta):
    xf = features.reshape(_B, _C, _N)
    xm = motion.reshape(_B, -1, _N)
    cm = xm.shape[1]
    xft = jnp.swapaxes(xf, 1, 2)
    xm8 = jnp.concatenate(
        [xm, jnp.zeros((_B, 8 - cm, _N), _F32)], axis=1)
    xmt = jnp.swapaxes(xm8, 1, 2)

    idx_a = _topk_call(xft, xf, _C, 0)
    idx_b = _topk_call(xmt, xm8, 8, 3)

    w1a_, w1b_ = w1a[:, :, 0, :], w1b[:, :, 0, :]
    taps = [w1a_[:, _C:, t].T for t in range(3)]
    taps += [w1b_[:, _C:, t].T for t in range(3)]
    a_a = jnp.sum(w1a_[:, :_C, :] + w1a_[:, _C:, :], axis=2).T
    a_b = jnp.sum(w1b_[:, :_C, :] + w1b_[:, _C:, :], axis=2).T
    wc = jnp.concatenate(taps + [a_a, a_b], axis=1)

    tbl, y1 = _tables_call(xft, wc)
    tblf = tbl.reshape(6 * _B * _N, _C)
    idx2 = jnp.concatenate(
        [idx_a.reshape(-1), idx_b.reshape(-1)]).reshape(_NW, -1, _IROW)

    g = _gather_rows(tblf, idx2).reshape(2, _B, 3, _N, _C)

    b1s = jnp.stack([b1a, b1b]).reshape(2, 1, _C)
    h, st1 = _h_call(y1, g, b1s)

    m1 = float(_B * _N * 3)
    mu1 = st1[:, 0] / m1
    var1 = st1[:, 1] / m1 - mu1 * mu1
    g1s = jnp.stack([g1a, g1b])
    be1s = jnp.stack([be1a, be1b])
    sc1 = g1s * lax.rsqrt(var1 + 1e-5)
    sh1 = be1s - mu1 * sc1

    w2c = jnp.stack([jnp.transpose(w2a[:, :, 0, :], (2, 1, 0)),
                     jnp.transpose(w2b[:, :, 0, :], (2, 1, 0))])
    b2s = jnp.stack([b2a, b2b]).reshape(2, 1, _C)
    h2, st2 = _conv2_call(h, sc1.reshape(2, 1, _C), sh1.reshape(2, 1, _C),
                          w2c, b2s)

    m2 = float(_B * _N)
    mu2 = st2[:, 0] / m2
    var2 = st2[:, 1] / m2 - mu2 * mu2
    g2s = jnp.stack([g2a, g2b])
    be2s = jnp.stack([be2a, be2b])
    sc2 = g2s * lax.rsqrt(var2 + 1e-5)
    sh2 = be2s - mu2 * sc2
    params = jnp.stack([sc2[0], sh2[0], sc2[1], sh2[1]])

    out = _final_call(h2, params, delta.reshape(1, 1))
    return out.reshape(_B, _C, _N, 1)


# speedup vs baseline: 15.5014x; 1.0703x over previous
import functools

import jax
import jax.numpy as jnp
from jax import lax
from jax.experimental import pallas as pl
from jax.experimental.pallas import tpu as pltpu
from jax.experimental.pallas import tpu_sc as plsc

_B, _C, _N, _K = 8, 128, 2000, 9
_RT = 400
_NT = _N // _RT
_F32 = jnp.float32

def _topk_body(xt_ref, x_ref, o_ref, *, tap_off):
    b = pl.program_id(0)
    xt = xt_ref[0]
    x = x_ref[0]
    d = jnp.dot(xt + xt, x, preferred_element_type=_F32)
    d = d - jnp.sum(x * x, axis=0, keepdims=True)
    lane_f = lax.broadcasted_iota(jnp.int32, d.shape, 1).astype(_F32)
    lane_k = lax.broadcasted_iota(jnp.int32, (_RT, _K), 1)
    out = jnp.zeros((_RT, _K), jnp.int32)
    for kk in range(_K):
        m = jnp.max(d, axis=1, keepdims=True)
        ge = d >= m
        amf = jnp.min(jnp.where(ge, lane_f, float(_N + 7)), axis=1,
                      keepdims=True)
        d = jnp.where(ge, -jnp.inf, d)
        out = jnp.where(lane_k == kk, amf.astype(jnp.int32), out)
    tap = (lane_k % 3 + tap_off) * (_B * _N)
    o_ref[0] = out + (tap + b * _N)


def _topk_call(xt, x, cp, tap_off):
    return pl.pallas_call(
        functools.partial(_topk_body, tap_off=tap_off),
        grid=(_B, _NT),
        in_specs=[
            pl.BlockSpec((1, _RT, cp), lambda b, i: (b, i, 0)),
            pl.BlockSpec((1, cp, _N), lambda b, i: (b, 0, 0)),
        ],
        out_specs=pl.BlockSpec((1, _RT, _K), lambda b, i: (b, i, 0)),
        out_shape=jax.ShapeDtypeStruct((_B, _N, _K), jnp.int32),
    )(xt, x)


def _tables_body(xt_ref, w_ref, tbl_ref, y1_ref):
    res = jnp.dot(xt_ref[0], w_ref[...], preferred_element_type=_F32)
    for t in range(6):
        tbl_ref[t, 0] = res[:, t * _C:(t + 1) * _C]
    y1_ref[0, 0] = res[:, 768:896]
    y1_ref[1, 0] = res[:, 896:]


def _tables_call(xt, wc):
    return pl.pallas_call(
        _tables_body,
        grid=(_B, _NT),
        in_specs=[
            pl.BlockSpec((1, _RT, _C), lambda b, i: (b, i, 0)),
            pl.BlockSpec((_C, 1024), lambda b, i: (0, 0)),
        ],
        out_specs=[
            pl.BlockSpec((6, 1, _RT, _C), lambda b, i: (0, b, i, 0)),
            pl.BlockSpec((2, 1, _RT, _C), lambda b, i: (0, b, i, 0)),
        ],
        out_shape=[
            jax.ShapeDtypeStruct((6, _B, _N, _C), _F32),
            jax.ShapeDtypeStruct((2, _B, _N, _C), _F32),
        ],
    )(xt, wc)


_NPOS = 2 * _B * _N
_NW = 32
_PW = _NPOS // _NW
_PCH = 40
_CNC = _PW // _PCH
_GROWS = _PCH * _K
_IROW = 120


def _gather_body(tbl_hbm, idx_hbm, out_hbm, idx_v, rows_v, out_v, sem0, sem1):
    c = lax.axis_index("c")
    s = lax.axis_index("s")
    wid = s * 2 + c
    pltpu.sync_copy(idx_hbm.at[wid], idx_v)
    sems = (sem0, sem1)

    def fire(ci, buf):
        for q in range(3):
            pltpu.make_async_copy(
                tbl_hbm.at[idx_v.at[ci * 3 + q]],
                rows_v.at[buf, pl.ds(q * _IROW, _IROW)], sems[buf]).start()

    def drain(ci, buf):
        for q in range(3):
            pltpu.make_async_copy(
                tbl_hbm.at[idx_v.at[ci * 3 + q]],
                rows_v.at[buf, pl.ds(q * _IROW, _IROW)], sems[buf]).wait()

    def process(ci, buf):
        def pos(nl, acc):
            base = nl * _K
            for j in range(3):
                r0 = base + 3 * j
                for cc in range(8):
                    sl = pl.ds(cc * 16, 16)
                    out_v[j, nl, sl] = (rows_v[buf, r0, sl]
                                        + rows_v[buf, r0 + 1, sl]
                                        + rows_v[buf, r0 + 2, sl])
            return acc

        lax.fori_loop(0, _PCH, pos, 0)
        p0 = wid * _PW + ci * _PCH
        obase = (p0 // _N) * (3 * _N) + (p0 % _N)
        for j in range(3):
            pltpu.sync_copy(out_v.at[j],
                            out_hbm.at[pl.ds(obase + j * _N, _PCH)])

    fire(0, 0)

    def pair(pi, carry):
        ci0 = pi * 2
        ci1 = ci0 + 1

        @pl.when(ci1 < _CNC)
        def _():
            fire(ci1, 1)

        drain(ci0, 0)
        process(ci0, 0)

        @pl.when(ci1 + 1 < _CNC)
        def _():
            fire(ci1 + 1, 0)

        @pl.when(ci1 < _CNC)
        def _():
            drain(ci1, 1)
            process(ci1, 1)

        return carry

    lax.fori_loop(0, (_CNC + 1) // 2, pair, 0)


_gather_rows = pl.kernel(
    _gather_body,
    out_type=jax.ShapeDtypeStruct((2 * _B * 3 * _N, _C), _F32),
    mesh=plsc.VectorSubcoreMesh(core_axis_name="c", subcore_axis_name="s"),
    scratch_types=[
        pltpu.VMEM((_NPOS * _K // _NW // _IROW, _IROW), jnp.int32),
        pltpu.VMEM((2, _GROWS, _C), _F32),
        pltpu.VMEM((3, _PCH, _C), _F32),
        pltpu.SemaphoreType.DMA,
        pltpu.SemaphoreType.DMA,
    ],
)


def _h_body(y1_ref, g_ref, b1_ref, h_ref, st_ref):
    b = pl.program_id(1)
    i = pl.program_id(2)
    base = y1_ref[0, 0] + b1_ref[0]
    g = g_ref[0, 0]
    s1 = jnp.zeros((1, _C), _F32)
    s2 = jnp.zeros((1, _C), _F32)
    for j in range(3):
        hj = base - g[j]
        h_ref[0, 0, j] = hj
        s1 = s1 + jnp.sum(hj, axis=0, keepdims=True)
        s2 = s2 + jnp.sum(hj * hj, axis=0, keepdims=True)
    acc = jnp.concatenate([s1, s2], axis=0)

    @pl.when(jnp.logical_and(b == 0, i == 0))
    def _():
        st_ref[0] = acc

    @pl.when(jnp.logical_or(b > 0, i > 0))
    def _():
        st_ref[0] = st_ref[0] + acc


def _h_call(y1, g, b1s):
    return pl.pallas_call(
        _h_body,
        grid=(2, _B, _NT),
        in_specs=[
            pl.BlockSpec((1, 1, _RT, _C), lambda r, b, i: (r, b, i, 0)),
            pl.BlockSpec((1, 1, 3, _RT, _C), lambda r, b, i: (r, b, 0, i, 0)),
            pl.BlockSpec((1, 1, _C), lambda r, b, i: (r, 0, 0)),
        ],
        out_specs=[
            pl.BlockSpec((1, 1, 3, _RT, _C), lambda r, b, i: (r, b, 0, i, 0)),
            pl.BlockSpec((1, 2, _C), lambda r, b, i: (r, 0, 0)),
        ],
        out_shape=[
            jax.ShapeDtypeStruct((2, _B, 3, _N, _C), _F32),
            jax.ShapeDtypeStruct((2, 2, _C), _F32),
        ],
    )(y1, g, b1s)


def _conv2_body(h_ref, sc_ref, sh_ref, w_ref, b2_ref, h2_ref, st_ref):
    b = pl.program_id(1)
    i = pl.program_id(2)
    scale = sc_ref[0]
    shift = sh_ref[0]
    tot = b2_ref[0] + jnp.zeros((_RT, _C), _F32)
    for j in range(3):
        r = jnp.maximum(h_ref[0, 0, j] * scale + shift, 0.0)
        tot = tot + jnp.dot(r, w_ref[0, j], preferred_element_type=_F32)
    h2_ref[0, 0] = tot
    s1 = jnp.sum(tot, axis=0, keepdims=True)
    s2 = jnp.sum(tot * tot, axis=0, keepdims=True)
    acc = jnp.concatenate([s1, s2], axis=0)

    @pl.when(jnp.logical_and(b == 0, i == 0))
    def _():
        st_ref[0] = acc

    @pl.when(jnp.logical_or(b > 0, i > 0))
    def _():
        st_ref[0] = st_ref[0] + acc


def _conv2_call(h, sc1, sh1, w2c, b2s):
    return pl.pallas_call(
        _conv2_body,
        grid=(2, _B, _NT),
        in_specs=[
            pl.BlockSpec((1, 1, 3, _RT, _C), lambda r, b, i: (r, b, 0, i, 0)),
            pl.BlockSpec((1, 1, _C), lambda r, b, i: (r, 0, 0)),
            pl.BlockSpec((1, 1, _C), lambda r, b, i: (r, 0, 0)),
            pl.BlockSpec((1, 3, _C, _C), lambda r, b, i: (r, 0, 0, 0)),
            pl.BlockSpec((1, 1, _C), lambda r, b, i: (r, 0, 0)),
        ],
        out_specs=[
            pl.BlockSpec((1, 1, _RT, _C), lambda r, b, i: (r, b, i, 0)),
            pl.BlockSpec((1, 2, _C), lambda r, b, i: (r, 0, 0)),
        ],
        out_shape=[
            jax.ShapeDtypeStruct((2, _B, _N, _C), _F32),
            jax.ShapeDtypeStruct((2, 2, _C), _F32),
        ],
    )(h, sc1, sh1, w2c, b2s)


def _final_body(h2_ref, p_ref, d_ref, o_ref):
    r1 = jnp.maximum(h2_ref[0, 0] * p_ref[0:1] + p_ref[1:2], 0.0)
    r2 = jnp.maximum(h2_ref[1, 0] * p_ref[2:3] + p_ref[3:4], 0.0)
    res = r1 + d_ref[0, 0] * r2
    o_ref[0] = res.T


def _final_call(h2, params, delta):
    return pl.pallas_call(
        _final_body,
        grid=(_B,),
        in_specs=[
            pl.BlockSpec((2, 1, _N, _C), lambda b: (0, b, 0, 0)),
            pl.BlockSpec((4, _C), lambda b: (0, 0)),
            pl.BlockSpec((1, 1), lambda b: (0, 0)),
        ],
        out_specs=pl.BlockSpec((1, _C, _N), lambda b: (b, 0, 0)),
        out_shape=jax.ShapeDtypeStruct((_B, _C, _N), _F32),
    )(h2, params, delta)


def kernel(features, motion, w1a, b1a, g1a, be1a, w2a, b2a, g2a, be2a,
           w1b, b1b, g1b, be1b, w2b, b2b, g2b, be2b, delta):
    xf = features.reshape(_B, _C, _N)
    xm = motion.reshape(_B, -1, _N)
    cm = xm.shape[1]
    xft = jnp.swapaxes(xf, 1, 2)
    xm8 = jnp.concatenate(
        [xm, jnp.zeros((_B, 8 - cm, _N), _F32)], axis=1)
    xmt = jnp.swapaxes(xm8, 1, 2)

    idx_a = _topk_call(xft, xf, _C, 0)
    idx_b = _topk_call(xmt, xm8, 8, 3)

    w1a_, w1b_ = w1a[:, :, 0, :], w1b[:, :, 0, :]
    taps = [w1a_[:, _C:, t].T for t in range(3)]
    taps += [w1b_[:, _C:, t].T for t in range(3)]
    a_a = jnp.sum(w1a_[:, :_C, :] + w1a_[:, _C:, :], axis=2).T
    a_b = jnp.sum(w1b_[:, :_C, :] + w1b_[:, _C:, :], axis=2).T
    wc = jnp.concatenate(taps + [a_a, a_b], axis=1)

    tbl, y1 = _tables_call(xft, wc)
    tblf = tbl.reshape(6 * _B * _N, _C)
    idx2 = jnp.concatenate(
        [idx_a.reshape(-1), idx_b.reshape(-1)]).reshape(_NW, -1, _IROW)

    g = _gather_rows(tblf, idx2).reshape(2, _B, 3, _N, _C)

    b1s = jnp.stack([b1a, b1b]).reshape(2, 1, _C)
    h, st1 = _h_call(y1, g, b1s)

    m1 = float(_B * _N * 3)
    mu1 = st1[:, 0] / m1
    var1 = st1[:, 1] / m1 - mu1 * mu1
    g1s = jnp.stack([g1a, g1b])
    be1s = jnp.stack([be1a, be1b])
    sc1 = g1s * lax.rsqrt(var1 + 1e-5)
    sh1 = be1s - mu1 * sc1

    w2c = jnp.stack([jnp.transpose(w2a[:, :, 0, :], (2, 1, 0)),
                     jnp.transpose(w2b[:, :, 0, :], (2, 1, 0))])
    b2s = jnp.stack([b2a, b2b]).reshape(2, 1, _C)
    h2, st2 = _conv2_call(h, sc1.reshape(2, 1, _C), sh1.reshape(2, 1, _C),
                          w2c, b2s)

    m2 = float(_B * _N)
    mu2 = st2[:, 0] / m2
    var2 = st2[:, 1] / m2 - mu2 * mu2
    g2s = jnp.stack([g2a, g2b])
    be2s = jnp.stack([be2a, be2b])
    sc2 = g2s * lax.rsqrt(var2 + 1e-5)
    sh2 = be2s - mu2 * sc2
    params = jnp.stack([sc2[0], sh2[0], sc2[1], sh2[1]])

    out = _final_call(h2, params, delta.reshape(1, 1))
    return out.reshape(_B, _C, _N, 1)


# speedup vs baseline: 16.1692x; 1.0431x over previous
import functools

import jax
import jax.numpy as jnp
from jax import lax
from jax.experimental import pallas as pl
from jax.experimental.pallas import tpu as pltpu
from jax.experimental.pallas import tpu_sc as plsc

_B, _C, _N, _K = 8, 128, 2000, 9
_RT = 400
_NT = _N // _RT
_F32 = jnp.float32

def _top9(d, out_shape_rows):
    lane_f = lax.broadcasted_iota(jnp.int32, d.shape, 1).astype(_F32)
    lane_k = lax.broadcasted_iota(jnp.int32, (out_shape_rows, _K), 1)
    out = jnp.zeros((out_shape_rows, _K), jnp.int32)
    for kk in range(_K):
        m = jnp.max(d, axis=1, keepdims=True)
        ge = d >= m
        amf = jnp.min(jnp.where(ge, lane_f, float(_N + 7)), axis=1,
                      keepdims=True)
        d = jnp.where(ge, -jnp.inf, d)
        out = jnp.where(lane_k == kk, amf.astype(jnp.int32), out)
    return out, lane_k


def _topka_body(xt_ref, x_ref, w_ref, o_ref, tbl_ref, y1_ref):
    b = pl.program_id(0)
    xt = xt_ref[0]
    x = x_ref[0]
    d = jnp.dot(xt + xt, x, preferred_element_type=_F32)
    d = d - jnp.sum(x * x, axis=0, keepdims=True)
    out, lane_k = _top9(d, _RT)
    tap = (lane_k % 3) * (_B * _N)
    o_ref[0] = out + (tap + b * _N)
    res = jnp.dot(xt, w_ref[...], preferred_element_type=_F32)
    for t in range(6):
        tbl_ref[t, 0] = res[:, t * _C:(t + 1) * _C]
    y1_ref[0, 0] = res[:, 768:896]
    y1_ref[1, 0] = res[:, 896:]


def _topka_call(xt, x, wc):
    return pl.pallas_call(
        _topka_body,
        grid=(_B, _NT),
        in_specs=[
            pl.BlockSpec((1, _RT, _C), lambda b, i: (b, i, 0)),
            pl.BlockSpec((1, _C, _N), lambda b, i: (b, 0, 0)),
            pl.BlockSpec((_C, 1024), lambda b, i: (0, 0)),
        ],
        out_specs=[
            pl.BlockSpec((1, _RT, _K), lambda b, i: (b, i, 0)),
            pl.BlockSpec((6, 1, _RT, _C), lambda b, i: (0, b, i, 0)),
            pl.BlockSpec((2, 1, _RT, _C), lambda b, i: (0, b, i, 0)),
        ],
        out_shape=[
            jax.ShapeDtypeStruct((_B, _N, _K), jnp.int32),
            jax.ShapeDtypeStruct((6, _B, _N, _C), _F32),
            jax.ShapeDtypeStruct((2, _B, _N, _C), _F32),
        ],
    )(xt, x, wc)


def _topkb_body(xt_ref, x_ref, o_ref):
    b = pl.program_id(0)
    d = jnp.dot(xt_ref[0] + xt_ref[0], x_ref[0], preferred_element_type=_F32)
    d = d - jnp.sum(x_ref[0] * x_ref[0], axis=0, keepdims=True)
    out, lane_k = _top9(d, _RT)
    tap = (lane_k % 3 + 3) * (_B * _N)
    o_ref[0] = out + (tap + b * _N)


def _topkb_call(xt, x, cp):
    return pl.pallas_call(
        _topkb_body,
        grid=(_B, _NT),
        in_specs=[
            pl.BlockSpec((1, _RT, cp), lambda b, i: (b, i, 0)),
            pl.BlockSpec((1, cp, _N), lambda b, i: (b, 0, 0)),
        ],
        out_specs=pl.BlockSpec((1, _RT, _K), lambda b, i: (b, i, 0)),
        out_shape=jax.ShapeDtypeStruct((_B, _N, _K), jnp.int32),
    )(xt, x)


_NPOS = 2 * _B * _N
_NW = 32
_PW = _NPOS // _NW
_PCH = 40
_CNC = _PW // _PCH
_GROWS = _PCH * _K
_IROW = 120


def _gather_body(tbl_hbm, idx_hbm, out_hbm, idx_v, rows_v, out_v, sem0, sem1):
    c = lax.axis_index("c")
    s = lax.axis_index("s")
    wid = s * 2 + c
    pltpu.sync_copy(idx_hbm.at[wid], idx_v)
    sems = (sem0, sem1)

    def fire(ci, buf):
        for q in range(3):
            pltpu.make_async_copy(
                tbl_hbm.at[idx_v.at[ci * 3 + q]],
                rows_v.at[buf, pl.ds(q * _IROW, _IROW)], sems[buf]).start()

    def drain(ci, buf):
        for q in range(3):
            pltpu.make_async_copy(
                tbl_hbm.at[idx_v.at[ci * 3 + q]],
                rows_v.at[buf, pl.ds(q * _IROW, _IROW)], sems[buf]).wait()

    def process(ci, buf):
        def pos(nl, acc):
            base = nl * _K
            for j in range(3):
                r0 = base + 3 * j
                for cc in range(8):
                    sl = pl.ds(cc * 16, 16)
                    out_v[j, nl, sl] = (rows_v[buf, r0, sl]
                                        + rows_v[buf, r0 + 1, sl]
                                        + rows_v[buf, r0 + 2, sl])
            return acc

        lax.fori_loop(0, _PCH, pos, 0)
        p0 = wid * _PW + ci * _PCH
        obase = (p0 // _N) * (3 * _N) + (p0 % _N)
        for j in range(3):
            pltpu.sync_copy(out_v.at[j],
                            out_hbm.at[pl.ds(obase + j * _N, _PCH)])

    fire(0, 0)

    def pair(pi, carry):
        ci0 = pi * 2
        ci1 = ci0 + 1

        @pl.when(ci1 < _CNC)
        def _():
            fire(ci1, 1)

        drain(ci0, 0)
        process(ci0, 0)

        @pl.when(ci1 + 1 < _CNC)
        def _():
            fire(ci1 + 1, 0)

        @pl.when(ci1 < _CNC)
        def _():
            drain(ci1, 1)
            process(ci1, 1)

        return carry

    lax.fori_loop(0, (_CNC + 1) // 2, pair, 0)


_gather_rows = pl.kernel(
    _gather_body,
    out_type=jax.ShapeDtypeStruct((2 * _B * 3 * _N, _C), _F32),
    mesh=plsc.VectorSubcoreMesh(core_axis_name="c", subcore_axis_name="s"),
    scratch_types=[
        pltpu.VMEM((_NPOS * _K // _NW // _IROW, _IROW), jnp.int32),
        pltpu.VMEM((2, _GROWS, _C), _F32),
        pltpu.VMEM((3, _PCH, _C), _F32),
        pltpu.SemaphoreType.DMA,
        pltpu.SemaphoreType.DMA,
    ],
)


def _h_body(y1_ref, g_ref, b1_ref, h_ref, st_ref):
    b = pl.program_id(1)
    i = pl.program_id(2)
    base = y1_ref[0, 0] + b1_ref[0]
    g = g_ref[0, 0]
    s1 = jnp.zeros((1, _C), _F32)
    s2 = jnp.zeros((1, _C), _F32)
    for j in range(3):
        hj = base - g[j]
        h_ref[0, 0, j] = hj
        s1 = s1 + jnp.sum(hj, axis=0, keepdims=True)
        s2 = s2 + jnp.sum(hj * hj, axis=0, keepdims=True)
    acc = jnp.concatenate([s1, s2], axis=0)

    @pl.when(jnp.logical_and(b == 0, i == 0))
    def _():
        st_ref[0] = acc

    @pl.when(jnp.logical_or(b > 0, i > 0))
    def _():
        st_ref[0] = st_ref[0] + acc


def _h_call(y1, g, b1s):
    return pl.pallas_call(
        _h_body,
        grid=(2, _B, _NT),
        in_specs=[
            pl.BlockSpec((1, 1, _RT, _C), lambda r, b, i: (r, b, i, 0)),
            pl.BlockSpec((1, 1, 3, _RT, _C), lambda r, b, i: (r, b, 0, i, 0)),
            pl.BlockSpec((1, 1, _C), lambda r, b, i: (r, 0, 0)),
        ],
        out_specs=[
            pl.BlockSpec((1, 1, 3, _RT, _C), lambda r, b, i: (r, b, 0, i, 0)),
            pl.BlockSpec((1, 2, _C), lambda r, b, i: (r, 0, 0)),
        ],
        out_shape=[
            jax.ShapeDtypeStruct((2, _B, 3, _N, _C), _F32),
            jax.ShapeDtypeStruct((2, 2, _C), _F32),
        ],
    )(y1, g, b1s)


def _conv2_body(h_ref, st1_ref, gb_ref, w_ref, b2_ref, h2_ref, st_ref):
    b = pl.program_id(1)
    i = pl.program_id(2)
    m1 = 1.0 / float(_B * _N * 3)
    mu = st1_ref[0, 0:1] * m1
    var = st1_ref[0, 1:2] * m1 - mu * mu
    scale = gb_ref[0, 0:1] * lax.rsqrt(var + 1e-5)
    shift = gb_ref[0, 1:2] - mu * scale
    tot = b2_ref[0] + jnp.zeros((_RT, _C), _F32)
    for j in range(3):
        r = jnp.maximum(h_ref[0, 0, j] * scale + shift, 0.0)
        tot = tot + jnp.dot(r, w_ref[0, j], preferred_element_type=_F32)
    h2_ref[0, 0] = tot
    s1 = jnp.sum(tot, axis=0, keepdims=True)
    s2 = jnp.sum(tot * tot, axis=0, keepdims=True)
    acc = jnp.concatenate([s1, s2], axis=0)

    @pl.when(jnp.logical_and(b == 0, i == 0))
    def _():
        st_ref[0] = acc

    @pl.when(jnp.logical_or(b > 0, i > 0))
    def _():
        st_ref[0] = st_ref[0] + acc


def _conv2_call(h, st1, gb1, w2c, b2s):
    return pl.pallas_call(
        _conv2_body,
        grid=(2, _B, _NT),
        in_specs=[
            pl.BlockSpec((1, 1, 3, _RT, _C), lambda r, b, i: (r, b, 0, i, 0)),
            pl.BlockSpec((1, 2, _C), lambda r, b, i: (r, 0, 0)),
            pl.BlockSpec((1, 2, _C), lambda r, b, i: (r, 0, 0)),
            pl.BlockSpec((1, 3, _C, _C), lambda r, b, i: (r, 0, 0, 0)),
            pl.BlockSpec((1, 1, _C), lambda r, b, i: (r, 0, 0)),
        ],
        out_specs=[
            pl.BlockSpec((1, 1, _RT, _C), lambda r, b, i: (r, b, i, 0)),
            pl.BlockSpec((1, 2, _C), lambda r, b, i: (r, 0, 0)),
        ],
        out_shape=[
            jax.ShapeDtypeStruct((2, _B, _N, _C), _F32),
            jax.ShapeDtypeStruct((2, 2, _C), _F32),
        ],
    )(h, st1, gb1, w2c, b2s)


def _final_body(h2_ref, st2_ref, gb_ref, d_ref, o_ref):
    m2 = 1.0 / float(_B * _N)
    res = None
    for r in range(2):
        mu = st2_ref[r, 0:1] * m2
        var = st2_ref[r, 1:2] * m2 - mu * mu
        scale = gb_ref[r, 0:1] * lax.rsqrt(var + 1e-5)
        shift = gb_ref[r, 1:2] - mu * scale
        rr = jnp.maximum(h2_ref[r, 0] * scale + shift, 0.0)
        res = rr if r == 0 else res + d_ref[0, 0] * rr
    o_ref[0] = res.T


def _final_call(h2, st2, gb2, delta):
    return pl.pallas_call(
        _final_body,
        grid=(_B,),
        in_specs=[
            pl.BlockSpec((2, 1, _N, _C), lambda b: (0, b, 0, 0)),
            pl.BlockSpec((2, 2, _C), lambda b: (0, 0, 0)),
            pl.BlockSpec((2, 2, _C), lambda b: (0, 0, 0)),
            pl.BlockSpec((1, 1), lambda b: (0, 0)),
        ],
        out_specs=pl.BlockSpec((1, _C, _N), lambda b: (b, 0, 0)),
        out_shape=jax.ShapeDtypeStruct((_B, _C, _N), _F32),
    )(h2, st2, gb2, delta)


def kernel(features, motion, w1a, b1a, g1a, be1a, w2a, b2a, g2a, be2a,
           w1b, b1b, g1b, be1b, w2b, b2b, g2b, be2b, delta):
    xf = features.reshape(_B, _C, _N)
    xm = motion.reshape(_B, -1, _N)
    cm = xm.shape[1]
    xft = jnp.swapaxes(xf, 1, 2)
    xm8 = jnp.concatenate(
        [xm, jnp.zeros((_B, 8 - cm, _N), _F32)], axis=1)
    xmt = jnp.swapaxes(xm8, 1, 2)

    w1a_, w1b_ = w1a[:, :, 0, :], w1b[:, :, 0, :]
    taps = [w1a_[:, _C:, t].T for t in range(3)]
    taps += [w1b_[:, _C:, t].T for t in range(3)]
    a_a = jnp.sum(w1a_[:, :_C, :] + w1a_[:, _C:, :], axis=2).T
    a_b = jnp.sum(w1b_[:, :_C, :] + w1b_[:, _C:, :], axis=2).T
    wc = jnp.concatenate(taps + [a_a, a_b], axis=1)

    idx_a, tbl, y1 = _topka_call(xft, xf, wc)
    idx_b = _topkb_call(xmt, xm8, 8)
    tblf = tbl.reshape(6 * _B * _N, _C)
    idx2 = jnp.concatenate(
        [idx_a.reshape(-1), idx_b.reshape(-1)]).reshape(_NW, -1, _IROW)

    g = _gather_rows(tblf, idx2).reshape(2, _B, 3, _N, _C)

    b1s = jnp.stack([b1a, b1b]).reshape(2, 1, _C)
    h, st1 = _h_call(y1, g, b1s)

    gb1 = jnp.stack([jnp.stack([g1a, be1a]), jnp.stack([g1b, be1b])])
    w2c = jnp.stack([jnp.transpose(w2a[:, :, 0, :], (2, 1, 0)),
                     jnp.transpose(w2b[:, :, 0, :], (2, 1, 0))])
    b2s = jnp.stack([b2a, b2b]).reshape(2, 1, _C)
    h2, st2 = _conv2_call(h, st1, gb1, w2c, b2s)

    gb2 = jnp.stack([jnp.stack([g2a, be2a]), jnp.stack([g2b, be2b])])
    out = _final_call(h2, st2, gb2, delta.reshape(1, 1))
    return out.reshape(_B, _C, _N, 1)


# speedup vs baseline: 16.3601x; 1.0118x over previous
import functools

import jax
import jax.numpy as jnp
from jax import lax
from jax.experimental import pallas as pl
from jax.experimental.pallas import tpu as pltpu
from jax.experimental.pallas import tpu_sc as plsc

_B, _C, _N, _K = 8, 128, 2000, 9
_RT = 400
_NT = _N // _RT
_F32 = jnp.float32

def _top9(d, out_shape_rows):
    lane_f = lax.broadcasted_iota(jnp.int32, d.shape, 1).astype(_F32)
    lane_k = lax.broadcasted_iota(jnp.int32, (out_shape_rows, _K), 1)
    out = jnp.zeros((out_shape_rows, _K), jnp.int32)
    for kk in range(_K):
        m = jnp.max(d, axis=1, keepdims=True)
        ge = d >= m
        amf = jnp.min(jnp.where(ge, lane_f, float(_N + 7)), axis=1,
                      keepdims=True)
        d = jnp.where(ge, -jnp.inf, d)
        out = jnp.where(lane_k == kk, amf.astype(jnp.int32), out)
    return out, lane_k


def _topka_body(xt_ref, x_ref, w_ref, o_ref, tbl_ref, y1_ref):
    b = pl.program_id(0)
    xt = xt_ref[0]
    x = x_ref[0]
    d = jnp.dot(xt + xt, x, preferred_element_type=_F32)
    d = d - jnp.sum(x * x, axis=0, keepdims=True)
    out, lane_k = _top9(d, _RT)
    tap = (lane_k % 3) * (_B * _N)
    o_ref[0] = out + (tap + b * _N)
    res = jnp.dot(xt, w_ref[...], preferred_element_type=_F32)
    for t in range(6):
        tbl_ref[t, 0] = res[:, t * _C:(t + 1) * _C]
    y1_ref[0, 0] = res[:, 768:896]
    y1_ref[1, 0] = res[:, 896:]


def _topka_call(xt, x, wc):
    return pl.pallas_call(
        _topka_body,
        grid=(_B, _NT),
        in_specs=[
            pl.BlockSpec((1, _RT, _C), lambda b, i: (b, i, 0)),
            pl.BlockSpec((1, _C, _N), lambda b, i: (b, 0, 0)),
            pl.BlockSpec((_C, 1024), lambda b, i: (0, 0)),
        ],
        out_specs=[
            pl.BlockSpec((1, _RT, _K), lambda b, i: (b, i, 0)),
            pl.BlockSpec((6, 1, _RT, _C), lambda b, i: (0, b, i, 0)),
            pl.BlockSpec((2, 1, _RT, _C), lambda b, i: (0, b, i, 0)),
        ],
        out_shape=[
            jax.ShapeDtypeStruct((_B, _N, _K), jnp.int32),
            jax.ShapeDtypeStruct((6, _B, _N, _C), _F32),
            jax.ShapeDtypeStruct((2, _B, _N, _C), _F32),
        ],
    )(xt, x, wc)


def _topkb_body(xt_ref, x_ref, o_ref):
    b = pl.program_id(0)
    d = jnp.dot(xt_ref[0] + xt_ref[0], x_ref[0], preferred_element_type=_F32)
    d = d - jnp.sum(x_ref[0] * x_ref[0], axis=0, keepdims=True)
    out, lane_k = _top9(d, _RT)
    tap = (lane_k % 3 + 3) * (_B * _N)
    o_ref[0] = out + (tap + b * _N)


def _topkb_call(xt, x, cp):
    return pl.pallas_call(
        _topkb_body,
        grid=(_B, _NT),
        in_specs=[
            pl.BlockSpec((1, _RT, cp), lambda b, i: (b, i, 0)),
            pl.BlockSpec((1, cp, _N), lambda b, i: (b, 0, 0)),
        ],
        out_specs=pl.BlockSpec((1, _RT, _K), lambda b, i: (b, i, 0)),
        out_shape=jax.ShapeDtypeStruct((_B, _N, _K), jnp.int32),
    )(xt, x)


_NPOS = 2 * _B * _N
_NW = 32
_PW = _NPOS // _NW
_PCH = 40
_CNC = _PW // _PCH
_GROWS = _PCH * _K
_IROW = 120


def _gather_body(tbl_hbm, idx_hbm, y1_hbm, out_hbm, st_hbm,
                 idx_v, rows_v, y1_v, out_v, st_v, sem0, sem1):
    c = lax.axis_index("c")
    s = lax.axis_index("s")
    wid = s * 2 + c
    pltpu.sync_copy(idx_hbm.at[wid], idx_v)
    sems = (sem0, sem1)

    def fire(ci, buf):
        for q in range(3):
            pltpu.make_async_copy(
                tbl_hbm.at[idx_v.at[ci * 3 + q]],
                rows_v.at[buf, pl.ds(q * _IROW, _IROW)], sems[buf]).start()

    def drain(ci, buf):
        for q in range(3):
            pltpu.make_async_copy(
                tbl_hbm.at[idx_v.at[ci * 3 + q]],
                rows_v.at[buf, pl.ds(q * _IROW, _IROW)], sems[buf]).wait()

    def process(ci, buf, acc):
        p0 = wid * _PW + ci * _PCH
        pltpu.sync_copy(y1_hbm.at[pl.ds(p0, _PCH)], y1_v)

        def pos(nl, a):
            a = list(a)
            base = nl * _K
            for cc in range(8):
                sl = pl.ds(cc * 16, 16)
                yv = y1_v[nl, sl]
                for j in range(3):
                    r0 = base + 3 * j
                    v = yv - (rows_v[buf, r0, sl] + rows_v[buf, r0 + 1, sl]
                              + rows_v[buf, r0 + 2, sl])
                    out_v[j, nl, sl] = v
                    a[cc] = a[cc] + v
                    a[8 + cc] = a[8 + cc] + v * v
            return tuple(a)

        acc = lax.fori_loop(0, _PCH, pos, acc)
        obase = (p0 // _N) * (3 * _N) + (p0 % _N)
        for j in range(3):
            pltpu.sync_copy(out_v.at[j],
                            out_hbm.at[pl.ds(obase + j * _N, _PCH)])
        return acc

    acc0 = tuple(jnp.zeros((16,), _F32) for _ in range(16))
    fire(0, 0)

    def pair(pi, acc):
        ci0 = pi * 2
        fire(ci0 + 1, 1)
        drain(ci0, 0)
        acc = process(ci0, 0, acc)
        fire(ci0 + 2, 0)
        drain(ci0 + 1, 1)
        return process(ci0 + 1, 1, acc)

    acc = lax.fori_loop(0, (_CNC - 1) // 2, pair, acc0)
    drain(_CNC - 1, 0)
    acc = process(_CNC - 1, 0, acc)
    for cc in range(8):
        st_v[0, pl.ds(cc * 16, 16)] = acc[cc]
        st_v[1, pl.ds(cc * 16, 16)] = acc[8 + cc]
    pltpu.sync_copy(st_v, st_hbm.at[wid])


_gather_rows = pl.kernel(
    _gather_body,
    out_type=[
        jax.ShapeDtypeStruct((2 * _B * 3 * _N, _C), _F32),
        jax.ShapeDtypeStruct((_NW, 2, _C), _F32),
    ],
    mesh=plsc.VectorSubcoreMesh(core_axis_name="c", subcore_axis_name="s"),
    scratch_types=[
        pltpu.VMEM((_NPOS * _K // _NW // _IROW, _IROW), jnp.int32),
        pltpu.VMEM((2, _GROWS, _C), _F32),
        pltpu.VMEM((_PCH, _C), _F32),
        pltpu.VMEM((3, _PCH, _C), _F32),
        pltpu.VMEM((2, _C), _F32),
        pltpu.SemaphoreType.DMA,
        pltpu.SemaphoreType.DMA,
    ],
)


def _conv2_body(h_ref, st1_ref, gb_ref, w_ref, h2_ref, st_ref):
    b = pl.program_id(1)
    i = pl.program_id(2)
    m1 = 1.0 / float(_B * _N * 3)
    st1 = jnp.sum(st1_ref[0], axis=0)
    mu = st1[0:1] * m1
    var = st1[1:2] * m1 - mu * mu
    scale = gb_ref[0, 0:1] * lax.rsqrt(var + 1e-5)
    shift = gb_ref[0, 1:2] - mu * scale
    tot = jnp.zeros((_RT, _C), _F32)
    for j in range(3):
        r = jnp.maximum(h_ref[0, 0, j] * scale + shift, 0.0)
        tot = tot + jnp.dot(r, w_ref[0, j], preferred_element_type=_F32)
    h2_ref[0, 0] = tot
    s1 = jnp.sum(tot, axis=0, keepdims=True)
    s2 = jnp.sum(tot * tot, axis=0, keepdims=True)
    acc = jnp.concatenate([s1, s2], axis=0)

    @pl.when(jnp.logical_and(b == 0, i == 0))
    def _():
        st_ref[0] = acc

    @pl.when(jnp.logical_or(b > 0, i > 0))
    def _():
        st_ref[0] = st_ref[0] + acc


def _conv2_call(h, st1, gb1, w2c):
    return pl.pallas_call(
        _conv2_body,
        grid=(2, _B, _NT),
        in_specs=[
            pl.BlockSpec((1, 1, 3, _RT, _C), lambda r, b, i: (r, b, 0, i, 0)),
            pl.BlockSpec((1, 16, 2, _C), lambda r, b, i: (r, 0, 0, 0)),
            pl.BlockSpec((1, 2, _C), lambda r, b, i: (r, 0, 0)),
            pl.BlockSpec((1, 3, _C, _C), lambda r, b, i: (r, 0, 0, 0)),
        ],
        out_specs=[
            pl.BlockSpec((1, 1, _RT, _C), lambda r, b, i: (r, b, i, 0)),
            pl.BlockSpec((1, 2, _C), lambda r, b, i: (r, 0, 0)),
        ],
        out_shape=[
            jax.ShapeDtypeStruct((2, _B, _N, _C), _F32),
            jax.ShapeDtypeStruct((2, 2, _C), _F32),
        ],
    )(h, st1, gb1, w2c)


def _final_body(h2_ref, st2_ref, gb_ref, d_ref, o_ref):
    m2 = 1.0 / float(_B * _N)
    res = None
    for r in range(2):
        mu = st2_ref[r, 0:1] * m2
        var = st2_ref[r, 1:2] * m2 - mu * mu
        scale = gb_ref[r, 0:1] * lax.rsqrt(var + 1e-5)
        shift = gb_ref[r, 1:2] - mu * scale
        rr = jnp.maximum(h2_ref[r, 0] * scale + shift, 0.0)
        res = rr if r == 0 else res + d_ref[0, 0] * rr
    o_ref[0] = res.T


def _final_call(h2, st2, gb2, delta):
    return pl.pallas_call(
        _final_body,
        grid=(_B,),
        in_specs=[
            pl.BlockSpec((2, 1, _N, _C), lambda b: (0, b, 0, 0)),
            pl.BlockSpec((2, 2, _C), lambda b: (0, 0, 0)),
            pl.BlockSpec((2, 2, _C), lambda b: (0, 0, 0)),
            pl.BlockSpec((1, 1), lambda b: (0, 0)),
        ],
        out_specs=pl.BlockSpec((1, _C, _N), lambda b: (b, 0, 0)),
        out_shape=jax.ShapeDtypeStruct((_B, _C, _N), _F32),
    )(h2, st2, gb2, delta)


def kernel(features, motion, w1a, b1a, g1a, be1a, w2a, b2a, g2a, be2a,
           w1b, b1b, g1b, be1b, w2b, b2b, g2b, be2b, delta):
    xf = features.reshape(_B, _C, _N)
    xm = motion.reshape(_B, -1, _N)
    cm = xm.shape[1]
    xft = jnp.swapaxes(xf, 1, 2)
    xm8 = jnp.concatenate(
        [xm, jnp.zeros((_B, 8 - cm, _N), _F32)], axis=1)
    xmt = jnp.swapaxes(xm8, 1, 2)

    w1a_, w1b_ = w1a[:, :, 0, :], w1b[:, :, 0, :]
    taps = [w1a_[:, _C:, t].T for t in range(3)]
    taps += [w1b_[:, _C:, t].T for t in range(3)]
    a_a = jnp.sum(w1a_[:, :_C, :] + w1a_[:, _C:, :], axis=2).T
    a_b = jnp.sum(w1b_[:, :_C, :] + w1b_[:, _C:, :], axis=2).T
    wc = jnp.concatenate(taps + [a_a, a_b], axis=1)

    idx_a, tbl, y1 = _topka_call(xft, xf, wc)
    idx_b = _topkb_call(xmt, xm8, 8)
    tblf = tbl.reshape(6 * _B * _N, _C)
    idx2 = jnp.concatenate(
        [idx_a.reshape(-1), idx_b.reshape(-1)]).reshape(_NW, -1, _IROW)

    y1f = y1.reshape(2 * _B * _N, _C)
    hf, st_raw = _gather_rows(tblf, idx2, y1f)
    h = hf.reshape(2, _B, 3, _N, _C)
    st1 = st_raw.reshape(2, 16, 2, _C)

    gb1 = jnp.stack([jnp.stack([g1a, be1a]), jnp.stack([g1b, be1b])])
    w2c = jnp.stack([jnp.transpose(w2a[:, :, 0, :], (2, 1, 0)),
                     jnp.transpose(w2b[:, :, 0, :], (2, 1, 0))])
    h2, st2 = _conv2_call(h, st1, gb1, w2c)

    gb2 = jnp.stack([jnp.stack([g2a, be2a]), jnp.stack([g2b, be2b])])
    out = _final_call(h2, st2, gb2, delta.reshape(1, 1))
    return out.reshape(_B, _C, _N, 1)


# speedup vs baseline: 16.3658x; 1.0004x over previous
import jax
import jax.numpy as jnp
from jax import lax
from jax.experimental import pallas as pl
from jax.experimental.pallas import tpu as pltpu
from jax.experimental.pallas import tpu_sc as plsc

_B, _C, _N, _K = 8, 128, 2000, 9
_RT = 400
_NT = _N // _RT
_F32 = jnp.float32

def _top9(d, out_shape_rows):
    lane_f = lax.broadcasted_iota(jnp.int32, d.shape, 1).astype(_F32)
    lane_k = lax.broadcasted_iota(jnp.int32, (out_shape_rows, _K), 1)
    out = jnp.zeros((out_shape_rows, _K), jnp.int32)
    for kk in range(_K):
        m = jnp.max(d, axis=1, keepdims=True)
        ge = d >= m
        amf = jnp.min(jnp.where(ge, lane_f, float(_N + 7)), axis=1,
                      keepdims=True)
        d = jnp.where(ge, -jnp.inf, d)
        out = jnp.where(lane_k == kk, amf.astype(jnp.int32), out)
    return out, lane_k


def _topka_body(xt_ref, x_ref, w_ref, o_ref, tbl_ref, y1_ref):
    b = pl.program_id(0)
    xt = xt_ref[0]
    x = x_ref[0]
    d = jnp.dot(xt + xt, x, preferred_element_type=_F32)
    d = d - jnp.sum(x * x, axis=0, keepdims=True)
    out, lane_k = _top9(d, _RT)
    tap = (lane_k % 3) * (_B * _N)
    o_ref[0] = out + (tap + b * _N)
    res = jnp.dot(xt, w_ref[...], preferred_element_type=_F32)
    for t in range(6):
        tbl_ref[t, 0] = res[:, t * _C:(t + 1) * _C]
    y1_ref[0, 0] = res[:, 768:896]
    y1_ref[1, 0] = res[:, 896:]


def _topka_call(xt, x, wc):
    return pl.pallas_call(
        _topka_body,
        grid=(_B, _NT),
        in_specs=[
            pl.BlockSpec((1, _RT, _C), lambda b, i: (b, i, 0)),
            pl.BlockSpec((1, _C, _N), lambda b, i: (b, 0, 0)),
            pl.BlockSpec((_C, 1024), lambda b, i: (0, 0)),
        ],
        out_specs=[
            pl.BlockSpec((1, _RT, _K), lambda b, i: (b, i, 0)),
            pl.BlockSpec((6, 1, _RT, _C), lambda b, i: (0, b, i, 0)),
            pl.BlockSpec((2, 1, _RT, _C), lambda b, i: (0, b, i, 0)),
        ],
        out_shape=[
            jax.ShapeDtypeStruct((_B, _N, _K), jnp.int32),
            jax.ShapeDtypeStruct((6, _B, _N, _C), _F32),
            jax.ShapeDtypeStruct((2, _B, _N, _C), _F32),
        ],
    )(xt, x, wc)


def _topkb_body(xt_ref, x_ref, o_ref):
    b = pl.program_id(0)
    d = jnp.dot(xt_ref[0] + xt_ref[0], x_ref[0], preferred_element_type=_F32)
    d = d - jnp.sum(x_ref[0] * x_ref[0], axis=0, keepdims=True)
    out, lane_k = _top9(d, _RT)
    tap = (lane_k % 3 + 3) * (_B * _N)
    o_ref[0] = out + (tap + b * _N)


def _topkb_call(xt, x, cp):
    return pl.pallas_call(
        _topkb_body,
        grid=(_B, _NT),
        in_specs=[
            pl.BlockSpec((1, _RT, cp), lambda b, i: (b, i, 0)),
            pl.BlockSpec((1, cp, _N), lambda b, i: (b, 0, 0)),
        ],
        out_specs=pl.BlockSpec((1, _RT, _K), lambda b, i: (b, i, 0)),
        out_shape=jax.ShapeDtypeStruct((_B, _N, _K), jnp.int32),
    )(xt, x)


_NPOS = 2 * _B * _N
_NW = 32
_PW = _NPOS // _NW
_PCH = 40
_CNC = _PW // _PCH
_GROWS = _PCH * _K
_IROW = 120


def _gather_body(tbl_hbm, idx_hbm, y1_hbm, out_hbm, st_hbm,
                 idx_v, rows_v, y1_v, out_v, st_v, sem0, sem1):
    c = lax.axis_index("c")
    s = lax.axis_index("s")
    wid = s * 2 + c
    pltpu.sync_copy(idx_hbm.at[wid], idx_v)
    sems = (sem0, sem1)

    def fire(ci, buf):
        for q in range(3):
            pltpu.make_async_copy(
                tbl_hbm.at[idx_v.at[ci * 3 + q]],
                rows_v.at[buf, pl.ds(q * _IROW, _IROW)], sems[buf]).start()

    def drain(ci, buf):
        for q in range(3):
            pltpu.make_async_copy(
                tbl_hbm.at[idx_v.at[ci * 3 + q]],
                rows_v.at[buf, pl.ds(q * _IROW, _IROW)], sems[buf]).wait()

    def process(ci, buf, acc):
        p0 = wid * _PW + ci * _PCH
        pltpu.sync_copy(y1_hbm.at[pl.ds(p0, _PCH)], y1_v)

        def pos(nl, a):
            a = list(a)
            base = nl * _K
            for cc in range(8):
                sl = pl.ds(cc * 16, 16)
                yv = y1_v[nl, sl]
                for j in range(3):
                    r0 = base + 3 * j
                    v = yv - (rows_v[buf, r0, sl] + rows_v[buf, r0 + 1, sl]
                              + rows_v[buf, r0 + 2, sl])
                    out_v[j, nl, sl] = v
                    a[cc] = a[cc] + v
                    a[8 + cc] = a[8 + cc] + v * v
            return tuple(a)

        acc = lax.fori_loop(0, _PCH, pos, acc)
        obase = (p0 // _N) * (3 * _N) + (p0 % _N)
        for j in range(3):
            pltpu.sync_copy(out_v.at[j],
                            out_hbm.at[pl.ds(obase + j * _N, _PCH)])
        return acc

    acc0 = tuple(jnp.zeros((16,), _F32) for _ in range(16))
    fire(0, 0)

    def pair(pi, acc):
        ci0 = pi * 2
        fire(ci0 + 1, 1)
        drain(ci0, 0)
        acc = process(ci0, 0, acc)
        fire(ci0 + 2, 0)
        drain(ci0 + 1, 1)
        return process(ci0 + 1, 1, acc)

    acc = lax.fori_loop(0, (_CNC - 1) // 2, pair, acc0)
    drain(_CNC - 1, 0)
    acc = process(_CNC - 1, 0, acc)
    for cc in range(8):
        st_v[0, pl.ds(cc * 16, 16)] = acc[cc]
        st_v[1, pl.ds(cc * 16, 16)] = acc[8 + cc]
    pltpu.sync_copy(st_v, st_hbm.at[wid])


_gather_rows = pl.kernel(
    _gather_body,
    out_type=[
        jax.ShapeDtypeStruct((2 * _B * 3 * _N, _C), _F32),
        jax.ShapeDtypeStruct((_NW, 2, _C), _F32),
    ],
    mesh=plsc.VectorSubcoreMesh(core_axis_name="c", subcore_axis_name="s"),
    scratch_types=[
        pltpu.VMEM((_NPOS * _K // _NW // _IROW, _IROW), jnp.int32),
        pltpu.VMEM((2, _GROWS, _C), _F32),
        pltpu.VMEM((_PCH, _C), _F32),
        pltpu.VMEM((3, _PCH, _C), _F32),
        pltpu.VMEM((2, _C), _F32),
        pltpu.SemaphoreType.DMA,
        pltpu.SemaphoreType.DMA,
    ],
)


def _conv2_body(h_ref, st1_ref, gb_ref, w_ref, h2_ref, st_ref):
    b = pl.program_id(1)
    i = pl.program_id(2)
    m1 = 1.0 / float(_B * _N * 3)
    st1 = jnp.sum(st1_ref[0], axis=0)
    mu = st1[0:1] * m1
    var = st1[1:2] * m1 - mu * mu
    scale = gb_ref[0, 0:1] * lax.rsqrt(var + 1e-5)
    shift = gb_ref[0, 1:2] - mu * scale
    tot = jnp.zeros((_RT, _C), _F32)
    for j in range(3):
        r = jnp.maximum(h_ref[0, 0, j] * scale + shift, 0.0)
        tot = tot + jnp.dot(r, w_ref[0, j], preferred_element_type=_F32)
    h2_ref[0, 0] = tot
    s1 = jnp.sum(tot, axis=0, keepdims=True)
    s2 = jnp.sum(tot * tot, axis=0, keepdims=True)
    acc = jnp.concatenate([s1, s2], axis=0)

    @pl.when(jnp.logical_and(b == 0, i == 0))
    def _():
        st_ref[0] = acc

    @pl.when(jnp.logical_or(b > 0, i > 0))
    def _():
        st_ref[0] = st_ref[0] + acc


def _conv2_call(h, st1, gb1, w2c):
    return pl.pallas_call(
        _conv2_body,
        grid=(2, _B, _NT),
        in_specs=[
            pl.BlockSpec((1, 1, 3, _RT, _C), lambda r, b, i: (r, b, 0, i, 0)),
            pl.BlockSpec((1, 16, 2, _C), lambda r, b, i: (r, 0, 0, 0)),
            pl.BlockSpec((1, 2, _C), lambda r, b, i: (r, 0, 0)),
            pl.BlockSpec((1, 3, _C, _C), lambda r, b, i: (r, 0, 0, 0)),
        ],
        out_specs=[
            pl.BlockSpec((1, 1, _RT, _C), lambda r, b, i: (r, b, i, 0)),
            pl.BlockSpec((1, 2, _C), lambda r, b, i: (r, 0, 0)),
        ],
        out_shape=[
            jax.ShapeDtypeStruct((2, _B, _N, _C), _F32),
            jax.ShapeDtypeStruct((2, 2, _C), _F32),
        ],
    )(h, st1, gb1, w2c)


def _final_body(h2_ref, st2_ref, gb_ref, d_ref, o_ref):
    m2 = 1.0 / float(_B * _N)
    res = None
    for r in range(2):
        mu = st2_ref[r, 0:1] * m2
        var = st2_ref[r, 1:2] * m2 - mu * mu
        scale = gb_ref[r, 0:1] * lax.rsqrt(var + 1e-5)
        shift = gb_ref[r, 1:2] - mu * scale
        rr = jnp.maximum(h2_ref[r, 0] * scale + shift, 0.0)
        res = rr if r == 0 else res + d_ref[0, 0] * rr
    o_ref[0] = res.T


def _final_call(h2, st2, gb2, delta):
    return pl.pallas_call(
        _final_body,
        grid=(_B,),
        in_specs=[
            pl.BlockSpec((2, 1, _N, _C), lambda b: (0, b, 0, 0)),
            pl.BlockSpec((2, 2, _C), lambda b: (0, 0, 0)),
            pl.BlockSpec((2, 2, _C), lambda b: (0, 0, 0)),
            pl.BlockSpec((1, 1), lambda b: (0, 0)),
        ],
        out_specs=pl.BlockSpec((1, _C, _N), lambda b: (b, 0, 0)),
        out_shape=jax.ShapeDtypeStruct((_B, _C, _N), _F32),
    )(h2, st2, gb2, delta)


def kernel(features, motion, w1a, b1a, g1a, be1a, w2a, b2a, g2a, be2a,
           w1b, b1b, g1b, be1b, w2b, b2b, g2b, be2b, delta):
    xf = features.reshape(_B, _C, _N)
    xm = motion.reshape(_B, -1, _N)
    cm = xm.shape[1]
    xft = jnp.swapaxes(xf, 1, 2)
    xm8 = jnp.concatenate(
        [xm, jnp.zeros((_B, 8 - cm, _N), _F32)], axis=1)
    xmt = jnp.swapaxes(xm8, 1, 2)

    w1a_, w1b_ = w1a[:, :, 0, :], w1b[:, :, 0, :]
    taps = [w1a_[:, _C:, t].T for t in range(3)]
    taps += [w1b_[:, _C:, t].T for t in range(3)]
    a_a = jnp.sum(w1a_[:, :_C, :] + w1a_[:, _C:, :], axis=2).T
    a_b = jnp.sum(w1b_[:, :_C, :] + w1b_[:, _C:, :], axis=2).T
    wc = jnp.concatenate(taps + [a_a, a_b], axis=1)

    idx_a, tbl, y1 = _topka_call(xft, xf, wc)
    idx_b = _topkb_call(xmt, xm8, 8)
    tblf = tbl.reshape(6 * _B * _N, _C)
    idx2 = jnp.concatenate(
        [idx_a.reshape(-1), idx_b.reshape(-1)]).reshape(_NW, -1, _IROW)

    y1f = y1.reshape(2 * _B * _N, _C)
    hf, st_raw = _gather_rows(tblf, idx2, y1f)
    h = hf.reshape(2, _B, 3, _N, _C)
    st1 = st_raw.reshape(2, 16, 2, _C)

    gb1 = jnp.stack([jnp.stack([g1a, be1a]), jnp.stack([g1b, be1b])])
    w2c = jnp.stack([jnp.transpose(w2a[:, :, 0, :], (2, 1, 0)),
                     jnp.transpose(w2b[:, :, 0, :], (2, 1, 0))])
    h2, st2 = _conv2_call(h, st1, gb1, w2c)

    gb2 = jnp.stack([jnp.stack([g2a, be2a]), jnp.stack([g2b, be2b])])
    out = _final_call(h2, st2, gb2, delta.reshape(1, 1))
    return out.reshape(_B, _C, _N, 1)
